# baseline (speedup 1.0000x reference)
import functools

import jax
import jax.numpy as jnp
from jax import lax
from jax.experimental import pallas as pl
from jax.experimental.pallas import tpu as pltpu
from jax.experimental.pallas import tpu_sc as plsc

B, T, D = 16, 2048, 1024
NC, NS, L = 2, 16, 16
BPC = B // NC
CH = 32
NBUF = 3
NCHUNK = T // CH
KMAX = NCHUNK // NS
NSLOT = BPC * KMAX
NJ = D // L
ROT = 3
AN, AD = 4, 5
BT = 256
NT = T // BT


def _tc_rows_of(lb):
    return CH * ((AN * lb) // (AD * CH))


def _sc_body(payload, seq_lens, out, buf0, buf1, buf2, acc, lens_v,
             tmp, res, shared, sem0, sem1, sem2):
    c = lax.axis_index("c")
    s = lax.axis_index("s")
    bufs = (buf0, buf1, buf2)
    sems = (sem0, sem1, sem2)

    pltpu.sync_copy(seq_lens, lens_v.at[pl.ds(0, B)])

    def len_of(b):
        return lens_v[pl.ds(b, L)][0]

    zero = jnp.zeros((L,), jnp.float32)

    def zero_body(i, carry):
        for j in range(NJ):
            acc[i, pl.ds(j * L, L)] = zero
        return carry

    lax.fori_loop(0, BPC, zero_body, 0)

    def decode(slot):
        i = slot // KMAX
        k = slot % KMAX
        b = BPC * c + i
        phase = (s + ROT * i) % NS
        row0 = CH * (phase + NS * k)
        lb = len_of(b)
        active = (row0 >= _tc_rows_of(lb)) & (row0 < lb)
        return i, b, row0, lb, active

    def start(slot, p):
        i, b, row0, lb, active = decode(slot)

        @pl.when(active)
        def _():
            pltpu.make_async_copy(
                payload.at[b, pl.ds(row0, CH), :], bufs[p], sems[p]).start()

    def process(slot, p):
        i, b, row0, lb, active = decode(slot)

        @pl.when(active)
        def _():
            pltpu.make_async_copy(
                payload.at[b, pl.ds(row0, CH), :], bufs[p], sems[p]).wait()
            nv = jnp.minimum(CH, lb - row0)
            buf = bufs[p]

            @pl.when(nv == CH)
            def _full():
                @plsc.parallel_loop(0, NJ, unroll=2)
                def _cols(j):
                    sl = pl.ds(j * L, L)
                    vs = [buf[r, sl] for r in range(CH)]
                    while len(vs) > 1:
                        vs = [a + bb for a, bb in zip(vs[::2], vs[1::2])]
                    plsc.addupdate(acc.at[i, sl], vs[0])

            @pl.when(nv < CH)
            def _partial():
                @plsc.parallel_loop(0, NJ, unroll=2)
                def _cols(j):
                    sl = pl.ds(j * L, L)
                    vs = [jnp.where(r < nv, buf[r, sl], zero)
                          for r in range(CH)]
                    while len(vs) > 1:
                        vs = [a + bb for a, bb in zip(vs[::2], vs[1::2])]
                    plsc.addupdate(acc.at[i, sl], vs[0])

    for p in range(NBUF - 1):
        start(p, p)

    ngroups = (NSLOT - (NBUF - 1)) // NBUF

    def main_body(g, carry):
        m = g * NBUF
        for pp in range(NBUF):
            slot = m + pp
            start(slot + NBUF - 1, (pp + NBUF - 1) % NBUF)
            process(slot, pp)
        return carry

    lax.fori_loop(0, ngroups, main_body, 0)
    for slot in range(ngroups * NBUF, NSLOT):
        process(slot, slot % NBUF)

    pltpu.sync_copy(acc, shared.at[s])
    plsc.subcore_barrier()

    @pl.when(s < BPC)
    def _():
        b = BPC * c + s
        pltpu.sync_copy(shared.at[0, s], res)

        def comb_body(w, carry):
            pltpu.sync_copy(shared.at[w, s], tmp)

            @plsc.parallel_loop(0, NJ, unroll=4)
            def _cols(j):
                sl = pl.ds(j * L, L)
                plsc.addupdate(res.at[sl], tmp[sl])

            return carry

        lax.fori_loop(1, NS, comb_body, 0)
        pltpu.sync_copy(res, out.at[b])


_sc_call = pl.kernel(
    _sc_body,
    out_type=jax.ShapeDtypeStruct((B, D), jnp.float32),
    mesh=plsc.VectorSubcoreMesh(core_axis_name="c", subcore_axis_name="s",
                                num_cores=NC, num_subcores=NS),
    scratch_types=[
        pltpu.VMEM((CH, D), jnp.float32),
        pltpu.VMEM((CH, D), jnp.float32),
        pltpu.VMEM((CH, D), jnp.float32),
        pltpu.VMEM((BPC, D), jnp.float32),
        pltpu.VMEM((B + L,), jnp.int32),
        pltpu.VMEM((D,), jnp.float32),
        pltpu.VMEM((D,), jnp.float32),
        pltpu.VMEM_SHARED((NS, BPC, D), jnp.float32),
        pltpu.SemaphoreType.DMA,
        pltpu.SemaphoreType.DMA,
        pltpu.SemaphoreType.DMA,
    ],
)


NBT = 8
NTSLOT = B * NT


def _tc_body(lens_ref, payload_a, payload_b, out, buf, acc, sems, osem):
    acc[...] = jnp.zeros_like(acc)
    payloads = (payload_a, payload_b)

    def decode(slot):
        b = slot // NT
        k = slot % NT
        lb = lens_ref[b]
        tc_rows = _tc_rows_of(lb)
        row0 = k * BT
        return b, row0, tc_rows

    def start(slot, p):
        b, row0, tc_rows = decode(slot)

        @pl.when(row0 < tc_rows)
        def _():
            pltpu.make_async_copy(
                payloads[p % 2].at[b, pl.ds(row0, BT), :], buf.at[p],
                sems.at[p]).start()

    def process(slot, p):
        b, row0, tc_rows = decode(slot)

        @pl.when(row0 < tc_rows)
        def _():
            pltpu.make_async_copy(
                payloads[p % 2].at[b, pl.ds(row0, BT), :], buf.at[p],
                sems.at[p]).wait()
            rows = row0 + lax.broadcasted_iota(jnp.int32, (BT, 1), 0)
            mask = (rows < tc_rows).astype(jnp.float32)
            psum = jnp.sum(buf[p] * mask, axis=0, keepdims=True)
            acc[pl.ds(b, 1), :] += psum

    for p in range(NBT - 1):
        start(p, p)

    ntg = (NTSLOT - (NBT - 1)) // NBT

    def main_body(g, carry):
        m = g * NBT
        for pp in range(NBT):
            slot = m + pp
            start(slot + NBT - 1, (pp + NBT - 1) % NBT)
            process(slot, pp)
        return carry

    lax.fori_loop(0, ntg, main_body, 0)
    for slot in range(ntg * NBT, NTSLOT):
        if slot + NBT - 1 < NTSLOT:
            start(slot + NBT - 1, (slot + NBT - 1) % NBT)
        process(slot, slot % NBT)

    cp = pltpu.make_async_copy(acc, out, osem)
    cp.start()
    cp.wait()


_tc_call = pl.pallas_call(
    _tc_body,
    grid_spec=pltpu.PrefetchScalarGridSpec(
        num_scalar_prefetch=1,
        in_specs=[pl.BlockSpec(memory_space=pltpu.MemorySpace.HBM),
                  pl.BlockSpec(memory_space=pltpu.MemorySpace.HBM)],
        out_specs=pl.BlockSpec(memory_space=pltpu.MemorySpace.HBM),
        scratch_shapes=[
            pltpu.VMEM((NBT, BT, D), jnp.float32),
            pltpu.VMEM((B, D), jnp.float32),
            pltpu.SemaphoreType.DMA((NBT,)),
            pltpu.SemaphoreType.DMA,
        ],
    ),
    out_shape=jax.ShapeDtypeStruct((B, D), jnp.float32),
)


def _comb_body(lens_ref, tc_ref, sc_ref, out_ref):
    for b in range(B):
        lbf = jnp.full((D,), lens_ref[b]).astype(jnp.float32)
        out_ref[b, :] = (tc_ref[b, :] + sc_ref[b, :]) / lbf


_comb_call = pl.pallas_call(
    _comb_body,
    in_specs=[pl.BlockSpec(memory_space=pltpu.MemorySpace.SMEM),
              pl.BlockSpec((B, D), lambda: (0, 0)),
              pl.BlockSpec((B, D), lambda: (0, 0))],
    out_specs=pl.BlockSpec((B, D), lambda: (0, 0)),
    out_shape=jax.ShapeDtypeStruct((B, D), jnp.float32),
)


@jax.jit
def kernel(payload, seq_lens):
    lens_i = seq_lens.astype(jnp.int32)
    sc_part = _sc_call(payload, lens_i)
    tc_part = _tc_call(lens_i, payload, payload)
    return _comb_call(lens_i, tc_part, sc_part)


# speedup vs baseline: 1.0531x; 1.0531x over previous
import functools

import jax
import jax.numpy as jnp
from jax import lax
from jax.experimental import pallas as pl
from jax.experimental.pallas import tpu as pltpu
from jax.experimental.pallas import tpu_sc as plsc

B, T, D = 16, 2048, 1024
NC, NS, L = 2, 16, 16
BPC = B // NC
CH = 32
NBUF = 3
NCHUNK = T // CH
KMAX = NCHUNK // NS
NSLOT = BPC * KMAX
NJ = D // L
ROT = 3
AN, AD = 4, 5
BT = 256
NT = T // BT


def _tc_rows_of(lb):
    return CH * ((AN * lb) // (AD * CH))


def _sc_body(payload, seq_lens, out, buf0, buf1, buf2, acc, lens_v,
             tmp, res, shared, sem0, sem1, sem2):
    c = lax.axis_index("c")
    s = lax.axis_index("s")
    bufs = (buf0, buf1, buf2)
    sems = (sem0, sem1, sem2)

    pltpu.sync_copy(seq_lens, lens_v.at[pl.ds(0, B)])

    def len_of(b):
        return lens_v[pl.ds(b, L)][0]

    zero = jnp.zeros((L,), jnp.float32)

    def zero_body(i, carry):
        for j in range(NJ):
            acc[i, pl.ds(j * L, L)] = zero
        return carry

    lax.fori_loop(0, BPC, zero_body, 0)

    def decode(slot):
        i = slot // KMAX
        k = slot % KMAX
        b = BPC * c + i
        phase = (s + ROT * i) % NS
        row0 = CH * (phase + NS * k)
        lb = len_of(b)
        active = (row0 >= _tc_rows_of(lb)) & (row0 < lb)
        return i, b, row0, lb, active

    def start(slot, p):
        i, b, row0, lb, active = decode(slot)

        @pl.when(active)
        def _():
            pltpu.make_async_copy(
                payload.at[b, pl.ds(row0, CH), :], bufs[p], sems[p]).start()

    def process(slot, p):
        i, b, row0, lb, active = decode(slot)

        @pl.when(active)
        def _():
            pltpu.make_async_copy(
                payload.at[b, pl.ds(row0, CH), :], bufs[p], sems[p]).wait()
            nv = jnp.minimum(CH, lb - row0)
            buf = bufs[p]

            @plsc.parallel_loop(0, NJ, unroll=1)
            def _cols(j):
                sl = pl.ds(j * L, L)
                vs = [jnp.where(r < nv, buf[r, sl], zero)
                      for r in range(CH)]
                while len(vs) > 1:
                    vs = [a + bb for a, bb in zip(vs[::2], vs[1::2])]
                plsc.addupdate(acc.at[i, sl], vs[0])

    for p in range(NBUF - 1):
        start(p, p)

    ngroups = (NSLOT - (NBUF - 1)) // NBUF

    def main_body(g, carry):
        m = g * NBUF
        for pp in range(NBUF):
            slot = m + pp
            start(slot + NBUF - 1, (pp + NBUF - 1) % NBUF)
            process(slot, pp)
        return carry

    lax.fori_loop(0, ngroups, main_body, 0)
    for slot in range(ngroups * NBUF, NSLOT):
        process(slot, slot % NBUF)

    pltpu.sync_copy(acc, shared.at[s])
    plsc.subcore_barrier()

    @pl.when(s < BPC)
    def _():
        b = BPC * c + s
        pltpu.sync_copy(shared.at[0, s], res)

        def comb_body(w, carry):
            pltpu.sync_copy(shared.at[w, s], tmp)

            @plsc.parallel_loop(0, NJ, unroll=4)
            def _cols(j):
                sl = pl.ds(j * L, L)
                plsc.addupdate(res.at[sl], tmp[sl])

            return carry

        lax.fori_loop(1, NS, comb_body, 0)
        pltpu.sync_copy(res, out.at[b])


_sc_call = pl.kernel(
    _sc_body,
    out_type=jax.ShapeDtypeStruct((B, D), jnp.float32),
    mesh=plsc.VectorSubcoreMesh(core_axis_name="c", subcore_axis_name="s",
                                num_cores=NC, num_subcores=NS),
    scratch_types=[
        pltpu.VMEM((CH, D), jnp.float32),
        pltpu.VMEM((CH, D), jnp.float32),
        pltpu.VMEM((CH, D), jnp.float32),
        pltpu.VMEM((BPC, D), jnp.float32),
        pltpu.VMEM((B + L,), jnp.int32),
        pltpu.VMEM((D,), jnp.float32),
        pltpu.VMEM((D,), jnp.float32),
        pltpu.VMEM_SHARED((NS, BPC, D), jnp.float32),
        pltpu.SemaphoreType.DMA,
        pltpu.SemaphoreType.DMA,
        pltpu.SemaphoreType.DMA,
    ],
)


NBT = 8
NTSLOT = B * NT


def _tc_body(lens_ref, payload_a, payload_b, out, buf, acc, sems, osem):
    acc[...] = jnp.zeros_like(acc)
    payloads = (payload_a, payload_b)

    def decode(slot):
        b = slot // NT
        k = slot % NT
        lb = lens_ref[b]
        tc_rows = _tc_rows_of(lb)
        row0 = k * BT
        return b, row0, tc_rows

    def start(slot, p):
        b, row0, tc_rows = decode(slot)

        @pl.when(row0 < tc_rows)
        def _():
            pltpu.make_async_copy(
                payloads[p % 2].at[b, pl.ds(row0, BT), :], buf.at[p],
                sems.at[p]).start()

    def process(slot, p):
        b, row0, tc_rows = decode(slot)

        @pl.when(row0 < tc_rows)
        def _():
            pltpu.make_async_copy(
                payloads[p % 2].at[b, pl.ds(row0, BT), :], buf.at[p],
                sems.at[p]).wait()
            rows = row0 + lax.broadcasted_iota(jnp.int32, (BT, 1), 0)
            mask = (rows < tc_rows).astype(jnp.float32)
            psum = jnp.sum(buf[p] * mask, axis=0, keepdims=True)
            acc[pl.ds(b, 1), :] += psum

    for p in range(NBT - 1):
        start(p, p)

    ntg = (NTSLOT - (NBT - 1)) // NBT

    def main_body(g, carry):
        m = g * NBT
        for pp in range(NBT):
            slot = m + pp
            start(slot + NBT - 1, (pp + NBT - 1) % NBT)
            process(slot, pp)
        return carry

    lax.fori_loop(0, ntg, main_body, 0)
    for slot in range(ntg * NBT, NTSLOT):
        if slot + NBT - 1 < NTSLOT:
            start(slot + NBT - 1, (slot + NBT - 1) % NBT)
        process(slot, slot % NBT)

    cp = pltpu.make_async_copy(acc, out, osem)
    cp.start()
    cp.wait()


_tc_call = pl.pallas_call(
    _tc_body,
    grid_spec=pltpu.PrefetchScalarGridSpec(
        num_scalar_prefetch=1,
        in_specs=[pl.BlockSpec(memory_space=pltpu.MemorySpace.HBM),
                  pl.BlockSpec(memory_space=pltpu.MemorySpace.HBM)],
        out_specs=pl.BlockSpec(memory_space=pltpu.MemorySpace.HBM),
        scratch_shapes=[
            pltpu.VMEM((NBT, BT, D), jnp.float32),
            pltpu.VMEM((B, D), jnp.float32),
            pltpu.SemaphoreType.DMA((NBT,)),
            pltpu.SemaphoreType.DMA,
        ],
    ),
    out_shape=jax.ShapeDtypeStruct((B, D), jnp.float32),
)


def _comb_body(lens_ref, tc_ref, sc_ref, out_ref):
    for b in range(B):
        lbf = jnp.full((D,), lens_ref[b]).astype(jnp.float32)
        out_ref[b, :] = (tc_ref[b, :] + sc_ref[b, :]) / lbf


_comb_call = pl.pallas_call(
    _comb_body,
    in_specs=[pl.BlockSpec(memory_space=pltpu.MemorySpace.SMEM),
              pl.BlockSpec((B, D), lambda: (0, 0)),
              pl.BlockSpec((B, D), lambda: (0, 0))],
    out_specs=pl.BlockSpec((B, D), lambda: (0, 0)),
    out_shape=jax.ShapeDtypeStruct((B, D), jnp.float32),
)


@jax.jit
def kernel(payload, seq_lens):
    lens_i = seq_lens.astype(jnp.int32)
    sc_part = _sc_call(payload, lens_i)
    tc_part = _tc_call(lens_i, payload, payload)
    return _comb_call(lens_i, tc_part, sc_part)
